# Initial kernel scaffold; baseline (speedup 1.0000x reference)
import jax
import jax.numpy as jnp
from jax.experimental import pallas as pl


def kernel(x, W, bias):
    raise NotImplementedError("write your pallas kernel here")


# speedup vs baseline: 1.2196x; 1.2196x over previous
import jax
import jax.numpy as jnp
from jax import lax
from jax.experimental import pallas as pl
from jax.experimental.pallas import tpu as pltpu
from jax.experimental.pallas import tpu_sc as plsc

_NUM_FIELDS = 26
_FIELD_W = 38462
_BATCH = 16384
_LANES = 16
_NC, _NS = 2, 16
_NW = _NC * _NS
_ROWS = _BATCH // _NW
_FLAT = _ROWS * _NUM_FIELDS


def _body(x_hbm, w_hbm, bias_hbm, out_hbm, idx_v, vals_v, out_v, bias_v, sem):
    wid = lax.axis_index("s") * _NC + lax.axis_index("c")
    fbase = wid * _FLAT

    pltpu.sync_copy(x_hbm.at[pl.ds(fbase, _FLAT)], idx_v)
    pltpu.sync_copy(bias_hbm, bias_v)

    def off_body(k, _):
        i16 = k * _LANES
        pos = i16 + lax.iota(jnp.int32, _LANES)
        f = lax.rem(pos, _NUM_FIELDS)
        idx_v[pl.ds(i16, _LANES)] = idx_v[pl.ds(i16, _LANES)] + f * _FIELD_W
        return 0

    lax.fori_loop(0, _FLAT // _LANES, off_body, 0)

    pltpu.async_copy(w_hbm.at[idx_v], vals_v, sem).wait()

    def red_body(c, _):
        j = c * _LANES + lax.iota(jnp.int32, _LANES)
        base_idx = j * _NUM_FIELDS

        def f_body(f, acc):
            return acc + plsc.load_gather(vals_v, [base_idx + f])

        acc = lax.fori_loop(0, _NUM_FIELDS, f_body, bias_v[...])
        out_v[pl.ds(c * _LANES, _LANES)] = acc
        return 0

    lax.fori_loop(0, _ROWS // _LANES, red_body, 0)

    pltpu.sync_copy(out_v, out_hbm.at[pl.ds(wid * _ROWS, _ROWS)])


def kernel(x, W, bias):
    x_flat = x.reshape(-1)
    w_flat = W.reshape(-1)
    bias16 = jnp.broadcast_to(bias, (_LANES,))

    run = pl.kernel(
        _body,
        out_type=jax.ShapeDtypeStruct((_BATCH,), jnp.float32),
        mesh=plsc.VectorSubcoreMesh(core_axis_name="c", subcore_axis_name="s"),
        compiler_params=pltpu.CompilerParams(needs_layout_passes=False),
        scratch_types=[
            pltpu.VMEM((_FLAT,), jnp.int32),
            pltpu.VMEM((_FLAT,), jnp.float32),
            pltpu.VMEM((_ROWS,), jnp.float32),
            pltpu.VMEM((_LANES,), jnp.float32),
            pltpu.SemaphoreType.DMA,
        ],
    )
    out = run(x_flat, w_flat, bias16)
    return out.reshape(_BATCH, 1)


# speedup vs baseline: 1.4528x; 1.1911x over previous
import jax
import jax.numpy as jnp
from jax import lax
from jax.experimental import pallas as pl
from jax.experimental.pallas import tpu as pltpu
from jax.experimental.pallas import tpu_sc as plsc

_NUM_FIELDS = 26
_FIELD_W = 38462
_SLAB = 38472
_BATCH = 16384
_LANES = 16
_NC, _NS = 2, 16
_HALF = _BATCH // _NC
_ROWS = _HALF // _NS
_PROWS, _PCOLS = 128, 64


def _body(x_hbm, w_hbm, bias_hbm, out_hbm,
          tab_a, tab_b, xcol_a, xcol_b, vals, bias_v, outb, tmp,
          shared_acc):
    c = lax.axis_index("c")
    s = lax.axis_index("s")
    has_b = s < (_NUM_FIELDS - _NS)
    rbase = c * _HALF

    fa = s
    ta = fa * _FIELD_W
    aligned_a = pl.multiple_of((ta // 8) * 8, 8)
    delta_a = ta - aligned_a
    pltpu.sync_copy(w_hbm.at[pl.ds(aligned_a, _SLAB)], tab_a)
    pltpu.sync_copy(x_hbm.at[fa, pl.ds(rbase, _HALF)], xcol_a)

    fb = s + _NS
    tb = fb * _FIELD_W
    aligned_b = pl.multiple_of((tb // 8) * 8, 8)
    delta_b = tb - aligned_b

    @pl.when(has_b)
    def _():
        pltpu.sync_copy(w_hbm.at[pl.ds(aligned_b, _SLAB)], tab_b)
        pltpu.sync_copy(x_hbm.at[fb, pl.ds(rbase, _HALF)], xcol_b)

    pltpu.sync_copy(bias_hbm, bias_v)

    def gat_a(r, _):
        for j in range(_PCOLS // _LANES):
            i = xcol_a[pl.ds(r * _PCOLS + j * _LANES, _LANES)]
            vals[r, pl.ds(j * _LANES, _LANES)] = plsc.load_gather(
                tab_a, [i + delta_a])
        return 0

    lax.fori_loop(0, _PROWS, gat_a, 0)

    @pl.when(has_b)
    def _():
        def gat_b(r, _):
            for j in range(_PCOLS // _LANES):
                i = xcol_b[pl.ds(r * _PCOLS + j * _LANES, _LANES)]
                vals[r, pl.ds(j * _LANES, _LANES)] = (
                    vals[r, pl.ds(j * _LANES, _LANES)]
                    + plsc.load_gather(tab_b, [i + delta_b]))
            return 0

        lax.fori_loop(0, _PROWS, gat_b, 0)

    pltpu.sync_copy(vals, shared_acc.at[s + (s >= 9).astype(jnp.int32)])
    plsc.subcore_barrier()

    pltpu.sync_copy(shared_acc.at[0, pl.ds(s * 8, 8)], outb)

    def acc_body(t, _):
        tslot = t + (t >= 9).astype(jnp.int32)
        pltpu.sync_copy(shared_acc.at[tslot, pl.ds(s * 8, 8)], tmp)
        for r in range(8):
            for j in range(_PCOLS // _LANES):
                outb[r, pl.ds(j * _LANES, _LANES)] = (
                    outb[r, pl.ds(j * _LANES, _LANES)]
                    + tmp[r, pl.ds(j * _LANES, _LANES)])
        return 0

    lax.fori_loop(1, _NS, acc_body, 0)

    def out_body(r, _):
        for j in range(_PCOLS // _LANES):
            outb[r, pl.ds(j * _LANES, _LANES)] = (
                outb[r, pl.ds(j * _LANES, _LANES)] + bias_v[...])
        return 0

    lax.fori_loop(0, 8, out_body, 0)
    pltpu.sync_copy(outb, out_hbm.at[c * _NS + s])


def kernel(x, W, bias):
    xt = x.T
    w_flat = jnp.pad(W.reshape(-1), (0, 20))
    bias16 = jnp.broadcast_to(bias, (_LANES,))

    run = pl.kernel(
        _body,
        out_type=jax.ShapeDtypeStruct((_NC * _NS, 8, _PCOLS), jnp.float32),
        mesh=plsc.VectorSubcoreMesh(core_axis_name="c", subcore_axis_name="s"),
        compiler_params=pltpu.CompilerParams(needs_layout_passes=False),
        scratch_types=[
            pltpu.VMEM((_SLAB,), jnp.float32),
            pltpu.VMEM((_SLAB,), jnp.float32),
            pltpu.VMEM((_HALF,), jnp.int32),
            pltpu.VMEM((_HALF,), jnp.int32),
            pltpu.VMEM((_PROWS, _PCOLS), jnp.float32),
            pltpu.VMEM((_LANES,), jnp.float32),
            pltpu.VMEM((8, _PCOLS), jnp.float32),
            pltpu.VMEM((8, _PCOLS), jnp.float32),
            pltpu.VMEM_SHARED((_NS + 2, _PROWS, _PCOLS), jnp.float32),
        ],
    )
    out = run(xt, w_flat, bias16)
    return out.reshape(_BATCH, 1)
